# baseline (speedup 1.0000x reference)
import jax
import jax.numpy as jnp
from jax import lax
from jax.experimental import pallas as pl
from jax.experimental.pallas import tpu as pltpu
from jax.experimental.pallas import tpu_sc as plsc

N_NODES = 10000
D = 128
DH = D // 2

NC = 2
NS = 16
LANES = 16

CHUNK = 128
NPAD = 10240
ZB = NPAD // NS
ZREP = ZB // CHUNK


def _pos_body(ewt_ref, wp1_ref, wp2_ref, out_ref):
    ew = ewt_ref[...].T
    h = jnp.maximum(
        jnp.dot(ew, wp1_ref[...], preferred_element_type=jnp.float32), 0.0)
    out_ref[...] = (
        jnp.dot(h, wp2_ref[...], preferred_element_type=jnp.float32)
        + 1.0).astype(jnp.bfloat16)


def _pos_call(ewt_pad, Wp1, Wp2, blk):
    e_pad = ewt_pad.shape[1]
    return pl.pallas_call(
        _pos_body,
        grid=(e_pad // blk,),
        in_specs=[
            pl.BlockSpec((2, blk), lambda i: (0, i)),
            pl.BlockSpec((2, D), lambda i: (0, 0)),
            pl.BlockSpec((D, D), lambda i: (0, 0)),
        ],
        out_specs=pl.BlockSpec((blk, D), lambda i: (i, 0)),
        out_shape=jax.ShapeDtypeStruct((e_pad, D), jnp.bfloat16),
    )(ewt_pad, Wp1, Wp2)


SUP = 8
DW = DH + LANES


def _sc_body(idx_hbm, pos_hbm, x_hbm,
             agg_out,
             isup0_v, isup1_v,
             pos0_v, pos1_v,
             xg0_v, xg1_v, xg2_v, xg3_v,
             msg0_v, msg1_v, agg_sh,
             sem_i0, sem_i1,
             sem_p0, sem_p1,
             sem_g0, sem_g1, sem_g2, sem_g3,
             sem_a0, sem_a1):
    c = lax.axis_index("c")
    s = lax.axis_index("s")
    isupb = (isup0_v, isup1_v)
    posb = (pos0_v, pos1_v)
    xgb = (xg0_v, xg1_v, xg2_v, xg3_v)
    msgb = (msg0_v, msg1_v)
    sem_i = (sem_i0, sem_i1)
    sem_p = (sem_p0, sem_p1)
    sem_g = (sem_g0, sem_g1, sem_g2, sem_g3)
    sem_a = (sem_a0, sem_a1)

    def _fill_zero(i, _):
        for k in range(DW // LANES):
            msg0_v[i, pl.ds(k * LANES, LANES)] = jnp.zeros((LANES,), jnp.float32)
        return 0
    lax.fori_loop(0, CHUNK, _fill_zero, 0)

    for r in range(ZREP):
        row0 = s * ZB + r * CHUNK
        pltpu.sync_copy(msg0_v, agg_sh.at[pl.ds(row0, CHUNK)])

    def _fill_ones(i, _):
        msg0_v[i, pl.ds(DH, LANES)] = jnp.full((LANES,), 1.0, jnp.float32)
        msg1_v[i, pl.ds(DH, LANES)] = jnp.full((LANES,), 1.0, jnp.float32)
        return 0
    lax.fori_loop(0, CHUNK, _fill_ones, 0)
    plsc.subcore_barrier()

    nsup_tot = idx_hbm.shape[0]
    nsup = nsup_tot // NS
    cpt = nsup * SUP
    base_g = s * nsup
    base_c = base_g * SUP
    x_c = x_hbm.at[c]
    col0 = c * (DH // 2)

    def _load_super(slot, g):
        pltpu.async_copy(idx_hbm.at[g], isupb[slot], sem_i[slot])

    def _wait_super(slot, g):
        pltpu.make_async_copy(idx_hbm.at[g], isupb[slot], sem_i[slot]).wait()

    def _issue_pos(pslot, ci):
        pltpu.async_copy(
            pos_hbm.at[pl.ds(ci * CHUNK, CHUNK), pl.ds(col0, DH // 2)],
            posb[pslot], sem_p[pslot])

    def _wait_pos(pslot, ci):
        pltpu.make_async_copy(
            pos_hbm.at[pl.ds(ci * CHUNK, CHUNK), pl.ds(col0, DH // 2)],
            posb[pslot], sem_p[pslot]).wait()

    def _issue_gather(gslot, src_ref):
        pltpu.async_copy(x_c.at[src_ref], xgb[gslot], sem_g[gslot])

    def _wait_gather(gslot, src_ref):
        pltpu.make_async_copy(x_c.at[src_ref], xgb[gslot], sem_g[gslot]).wait()

    def _issue_scatter(mslot, dst_ref):
        pltpu.async_copy(msgb[mslot], agg_sh.at[dst_ref], sem_a[mslot],
                         add=True)

    def _wait_scatter(mslot, dst_ref):
        pltpu.make_async_copy(msgb[mslot], agg_sh.at[dst_ref],
                              sem_a[mslot]).wait()

    _load_super(0, base_g)
    _wait_super(0, base_g)
    _issue_gather(0, isup0_v.at[0])
    _issue_gather(1, isup0_v.at[1])
    _issue_pos(0, base_c)

    def _spair(g0, _):
        for gg in range(2):
            jbase = (g0 * 2 + gg) * SUP
            sg = gg
            nsg = 1 - gg
            for u in range(SUP):
                j = jbase + u
                ci = base_c + j
                gslot = u % 4
                mslot = u % 2
                src_j = isupb[sg].at[u]
                dst_j = isupb[sg].at[SUP + u]

                if u == 2:
                    @pl.when(j + SUP < cpt)
                    def _():
                        _load_super(nsg, base_g + (g0 * 2 + gg) + 1)
                if u == 6:
                    @pl.when(j + 2 < cpt)
                    def _():
                        _wait_super(nsg, base_g + (g0 * 2 + gg) + 1)

                if u < 6:
                    sref = isupb[sg].at[u + 2]
                else:
                    sref = isupb[nsg].at[u - 6]

                @pl.when(j + 2 < cpt)
                def _():
                    _issue_gather((u + 2) % 4, sref)

                @pl.when(j + 1 < cpt)
                def _():
                    _issue_pos((u + 1) % 2, ci + 1)

                if u < 2:
                    pdst = isupb[nsg].at[SUP + u + 6]

                    @pl.when(j >= 2)
                    def _():
                        _wait_scatter(mslot, pdst)
                else:
                    _wait_scatter(mslot, isupb[sg].at[SUP + u - 2])

                _wait_pos(mslot, ci)
                _wait_gather(gslot, src_j)

                @plsc.parallel_loop(0, CHUNK, step=1, unroll=8)
                def _(i):
                    for g2 in range(DH // 32):
                        wp = posb[mslot][i, pl.ds(g2 * LANES, LANES)]
                        wx = xgb[gslot][i, pl.ds(g2 * LANES, LANES)]
                        pe = lax.bitcast_convert_type(wp << 16, jnp.float32)
                        po = lax.bitcast_convert_type(
                            wp & jnp.int32(-65536), jnp.float32)
                        xe = lax.bitcast_convert_type(wx << 16, jnp.float32)
                        xo = lax.bitcast_convert_type(
                            wx & jnp.int32(-65536), jnp.float32)
                        msgb[mslot][i, pl.ds(g2 * 32, LANES)] = pe * xe
                        msgb[mslot][i, pl.ds(g2 * 32 + LANES, LANES)] = po * xo

                _issue_scatter(mslot, dst_j)
        return 0
    lax.fori_loop(0, nsup // 2, _spair, 0)

    _wait_scatter(0, isup1_v.at[SUP + 6])
    _wait_scatter(1, isup1_v.at[SUP + 7])
    plsc.subcore_barrier()

    for r in range(ZREP):
        row0 = s * ZB + r * CHUNK
        pltpu.sync_copy(agg_sh.at[pl.ds(row0, CHUNK)],
                        agg_out.at[c, pl.ds(row0, CHUNK)])


def _sc_call(idx8, pos1, x_split):
    mesh = plsc.VectorSubcoreMesh(core_axis_name="c", subcore_axis_name="s")
    f = pl.kernel(
        _sc_body,
        out_type=jax.ShapeDtypeStruct((NC, NPAD, DW), jnp.float32),
        mesh=mesh,
        compiler_params=pltpu.CompilerParams(use_tc_tiling_on_sc=False),
        scratch_types=(
            [pltpu.VMEM((2 * SUP, CHUNK), jnp.int32)] * 2
            + [pltpu.VMEM((CHUNK, DH // 2), jnp.int32)] * 2
            + [pltpu.VMEM((CHUNK, DH // 2), jnp.int32)] * 4
            + [pltpu.VMEM((CHUNK, DW), jnp.float32)] * 2
            + [pltpu.VMEM_SHARED((NPAD, DW), jnp.float32)]
            + [pltpu.SemaphoreType.DMA] * 10
        ),
    )
    return f(idx8, pos1, x_split)


def _final_body(p_ref, x_ref, w_ref, b_ref, g_ref, be_ref, out_ref):
    xv = x_ref[...]
    agg = jnp.concatenate(
        [p_ref[0, :N_NODES, :DH], p_ref[1, :N_NODES, :DH]], axis=1) + 2.0 * xv
    cnt = p_ref[0, :N_NODES, DH:DH + 1] + 1.0
    agg = agg / cnt
    o = jnp.dot(agg, w_ref[...], preferred_element_type=jnp.float32) + b_ref[...]
    mean = jnp.mean(o, axis=0, keepdims=True)
    var = jnp.mean((o - mean) * (o - mean), axis=0, keepdims=True)
    o = (o - mean) * lax.rsqrt(var + 1e-5) * g_ref[...] + be_ref[...]
    out_ref[...] = jnp.maximum(o, 0.0) + xv


def _final_call(aggp, x, W, b, gamma, beta):
    return pl.pallas_call(
        _final_body,
        out_shape=jax.ShapeDtypeStruct((N_NODES, D), jnp.float32),
    )(aggp, x, W, b.reshape(1, D), gamma.reshape(1, D), beta.reshape(1, D))


def kernel(x, edge_index, edge_w, Wp1, Wp2, W, b, gamma, beta):
    src = edge_index[0]
    dst = edge_index[1]
    e = src.shape[0]
    span = CHUNK * NS * 2 * SUP
    e_pad = ((e + span - 1) // span) * span

    pad = e_pad - e
    src_p = jnp.concatenate([src, jnp.zeros((pad,), src.dtype)])
    dst_p = jnp.concatenate([dst, jnp.full((pad,), N_NODES, dst.dtype)])
    ewt_p = jnp.concatenate(
        [edge_w.T, jnp.zeros((2, pad), edge_w.dtype)], axis=1)
    perm = [c2 * DH + g2 * 32 + (t // 2 if t % 2 == 0 else 16 + t // 2)
            for c2 in range(2) for g2 in range(DH // 32) for t in range(32)]
    perm = jnp.array(perm, dtype=jnp.int32)
    xp = x[:, perm].astype(jnp.bfloat16)
    xpi = jax.lax.bitcast_convert_type(
        xp.reshape(N_NODES, D // 2, 2), jnp.int32)
    x_split = jnp.stack([xpi[:, :DH // 2], xpi[:, DH // 2:]], axis=0)
    Wp2p = Wp2[:, perm]
    idx8 = jnp.concatenate([src_p.reshape(-1, SUP, CHUNK),
                            dst_p.reshape(-1, SUP, CHUNK)], axis=1)

    pos1 = _pos_call(ewt_p, Wp1, Wp2p, blk=4096)
    pos1i = jax.lax.bitcast_convert_type(
        pos1.reshape(e_pad, D // 2, 2), jnp.int32)
    aggp = _sc_call(idx8, pos1i, x_split)
    return _final_call(aggp, x, W, b, gamma, beta)


# speedup vs baseline: 3.0299x; 3.0299x over previous
import jax
import jax.numpy as jnp
from jax import lax
from jax.experimental import pallas as pl
from jax.experimental.pallas import tpu as pltpu
from jax.experimental.pallas import tpu_sc as plsc

N_NODES = 10000
D = 128
DH = D // 2

NC = 2
NS = 16
LANES = 16

CHUNK = 128
NPAD = 10240
ZB = NPAD // NS
ZREP = ZB // CHUNK


def _pos_body(ewt_ref, wp1_ref, wp2_ref, out_ref):
    ew = ewt_ref[...].T
    h = jnp.maximum(
        jnp.dot(ew, wp1_ref[...], preferred_element_type=jnp.float32), 0.0)
    pv = jnp.dot(h, wp2_ref[...], preferred_element_type=jnp.float32) + 1.0
    lo = lax.bitcast_convert_type(
        pv[:, :DH].astype(jnp.bfloat16), jnp.uint16).astype(jnp.uint32)
    hi = lax.bitcast_convert_type(
        pv[:, DH:].astype(jnp.bfloat16), jnp.uint16).astype(jnp.uint32)
    packed = lax.bitcast_convert_type((hi << 16) | lo, jnp.int32)
    out_ref[...] = jnp.concatenate([packed, packed], axis=1)


def _pos_call(ewt_pad, Wp1, Wp2, blk):
    e_pad = ewt_pad.shape[1]
    return pl.pallas_call(
        _pos_body,
        grid=(e_pad // blk,),
        in_specs=[
            pl.BlockSpec((2, blk), lambda i: (0, i)),
            pl.BlockSpec((2, D), lambda i: (0, 0)),
            pl.BlockSpec((D, D), lambda i: (0, 0)),
        ],
        out_specs=pl.BlockSpec((blk, D), lambda i: (i, 0)),
        out_shape=jax.ShapeDtypeStruct((e_pad, D), jnp.int32),
    )(ewt_pad, Wp1, Wp2)


SUP = 8
DW = DH + LANES


def _sc_body(idx_hbm, pos_hbm, x_hbm,
             agg_out,
             isup0_v, isup1_v,
             pos0_v, pos1_v,
             xg0_v, xg1_v, xg2_v, xg3_v,
             msg0_v, msg1_v, agg_sh,
             sem_i0, sem_i1,
             sem_p0, sem_p1,
             sem_g0, sem_g1, sem_g2, sem_g3,
             sem_a0, sem_a1):
    c = lax.axis_index("c")
    s = lax.axis_index("s")
    isupb = (isup0_v, isup1_v)
    posb = (pos0_v, pos1_v)
    xgb = (xg0_v, xg1_v, xg2_v, xg3_v)
    msgb = (msg0_v, msg1_v)
    sem_i = (sem_i0, sem_i1)
    sem_p = (sem_p0, sem_p1)
    sem_g = (sem_g0, sem_g1, sem_g2, sem_g3)
    sem_a = (sem_a0, sem_a1)

    def _fill_zero(i, _):
        for k in range(DW // LANES):
            msg0_v[i, pl.ds(k * LANES, LANES)] = jnp.zeros((LANES,), jnp.float32)
        return 0
    lax.fori_loop(0, CHUNK, _fill_zero, 0)

    for r in range(ZREP):
        row0 = s * ZB + r * CHUNK
        pltpu.sync_copy(msg0_v, agg_sh.at[pl.ds(row0, CHUNK)])

    def _fill_ones(i, _):
        msg0_v[i, pl.ds(DH, LANES)] = jnp.full((LANES,), 1.0, jnp.float32)
        msg1_v[i, pl.ds(DH, LANES)] = jnp.full((LANES,), 1.0, jnp.float32)
        return 0
    lax.fori_loop(0, CHUNK, _fill_ones, 0)
    plsc.subcore_barrier()

    nsup_tot = idx_hbm.shape[0]
    nsup = nsup_tot // NS
    cpt = nsup * SUP
    base_g = s * nsup
    base_c = base_g * SUP
    x_c = x_hbm.at[c]
    col0 = c * (DH // 2)

    def _load_super(slot, g):
        pltpu.async_copy(idx_hbm.at[g], isupb[slot], sem_i[slot])

    def _wait_super(slot, g):
        pltpu.make_async_copy(idx_hbm.at[g], isupb[slot], sem_i[slot]).wait()

    def _issue_pos(pslot, ci):
        pltpu.async_copy(
            pos_hbm.at[pl.ds(ci * CHUNK, CHUNK), pl.ds(col0, DH // 2)],
            posb[pslot], sem_p[pslot])

    def _wait_pos(pslot, ci):
        pltpu.make_async_copy(
            pos_hbm.at[pl.ds(ci * CHUNK, CHUNK), pl.ds(col0, DH // 2)],
            posb[pslot], sem_p[pslot]).wait()

    def _issue_gather(gslot, src_ref):
        pltpu.async_copy(x_c.at[src_ref], xgb[gslot], sem_g[gslot])

    def _wait_gather(gslot, src_ref):
        pltpu.make_async_copy(x_c.at[src_ref], xgb[gslot], sem_g[gslot]).wait()

    def _issue_scatter(mslot, dst_ref):
        pltpu.async_copy(msgb[mslot], agg_sh.at[dst_ref], sem_a[mslot],
                         add=True)

    def _wait_scatter(mslot, dst_ref):
        pltpu.make_async_copy(msgb[mslot], agg_sh.at[dst_ref],
                              sem_a[mslot]).wait()

    _load_super(0, base_g)
    _wait_super(0, base_g)
    _issue_gather(0, isup0_v.at[0])
    _issue_gather(1, isup0_v.at[1])
    _issue_pos(0, base_c)

    def _spair(g0, _):
        for gg in range(2):
            jbase = (g0 * 2 + gg) * SUP
            sg = gg
            nsg = 1 - gg
            for u in range(SUP):
                j = jbase + u
                ci = base_c + j
                gslot = u % 4
                mslot = u % 2
                src_j = isupb[sg].at[u]
                dst_j = isupb[sg].at[SUP + u]

                if u == 2:
                    @pl.when(j + SUP < cpt)
                    def _():
                        _load_super(nsg, base_g + (g0 * 2 + gg) + 1)
                if u == 6:
                    @pl.when(j + 2 < cpt)
                    def _():
                        _wait_super(nsg, base_g + (g0 * 2 + gg) + 1)

                if u < 6:
                    sref = isupb[sg].at[u + 2]
                else:
                    sref = isupb[nsg].at[u - 6]

                @pl.when(j + 2 < cpt)
                def _():
                    _issue_gather((u + 2) % 4, sref)

                @pl.when(j + 1 < cpt)
                def _():
                    _issue_pos((u + 1) % 2, ci + 1)

                if u < 2:
                    pdst = isupb[nsg].at[SUP + u + 6]

                    @pl.when(j >= 2)
                    def _():
                        _wait_scatter(mslot, pdst)
                else:
                    _wait_scatter(mslot, isupb[sg].at[SUP + u - 2])

                _wait_pos(mslot, ci)
                _wait_gather(gslot, src_j)

                @plsc.parallel_loop(0, CHUNK, step=1, unroll=8)
                def _(i):
                    for g2 in range(DH // 32):
                        wp = posb[mslot][i, pl.ds(g2 * LANES, LANES)]
                        wx = xgb[gslot][i, pl.ds(g2 * LANES, LANES)]
                        pe = lax.bitcast_convert_type(wp << 16, jnp.float32)
                        po = lax.bitcast_convert_type(
                            wp & jnp.int32(-65536), jnp.float32)
                        xe = lax.bitcast_convert_type(wx << 16, jnp.float32)
                        xo = lax.bitcast_convert_type(
                            wx & jnp.int32(-65536), jnp.float32)
                        msgb[mslot][i, pl.ds(g2 * 32, LANES)] = pe * xe
                        msgb[mslot][i, pl.ds(g2 * 32 + LANES, LANES)] = po * xo

                _issue_scatter(mslot, dst_j)
        return 0
    lax.fori_loop(0, nsup // 2, _spair, 0)

    _wait_scatter(0, isup1_v.at[SUP + 6])
    _wait_scatter(1, isup1_v.at[SUP + 7])
    plsc.subcore_barrier()

    for r in range(ZREP):
        row0 = s * ZB + r * CHUNK
        pltpu.sync_copy(agg_sh.at[pl.ds(row0, CHUNK)],
                        agg_out.at[c, pl.ds(row0, CHUNK)])


def _sc_call(idx8, pos1, x_split):
    mesh = plsc.VectorSubcoreMesh(core_axis_name="c", subcore_axis_name="s")
    f = pl.kernel(
        _sc_body,
        out_type=jax.ShapeDtypeStruct((NC, NPAD, DW), jnp.float32),
        mesh=mesh,
        compiler_params=pltpu.CompilerParams(use_tc_tiling_on_sc=False),
        scratch_types=(
            [pltpu.VMEM((2 * SUP, CHUNK), jnp.int32)] * 2
            + [pltpu.VMEM((CHUNK, DH // 2), jnp.int32)] * 2
            + [pltpu.VMEM((CHUNK, DH // 2), jnp.int32)] * 4
            + [pltpu.VMEM((CHUNK, DW), jnp.float32)] * 2
            + [pltpu.VMEM_SHARED((NPAD, DW), jnp.float32)]
            + [pltpu.SemaphoreType.DMA] * 10
        ),
    )
    return f(idx8, pos1, x_split)


def _final_body(p_ref, x_ref, w_ref, b_ref, g_ref, be_ref, out_ref):
    xv = x_ref[...]
    agg = jnp.concatenate(
        [p_ref[0, :N_NODES, :DH], p_ref[1, :N_NODES, :DH]], axis=1) + 2.0 * xv
    cnt = p_ref[0, :N_NODES, DH:DH + 1] + 1.0
    agg = agg / cnt
    o = jnp.dot(agg, w_ref[...], preferred_element_type=jnp.float32) + b_ref[...]
    mean = jnp.mean(o, axis=0, keepdims=True)
    var = jnp.mean((o - mean) * (o - mean), axis=0, keepdims=True)
    o = (o - mean) * lax.rsqrt(var + 1e-5) * g_ref[...] + be_ref[...]
    out_ref[...] = jnp.maximum(o, 0.0) + xv


def _final_call(aggp, x, W, b, gamma, beta):
    return pl.pallas_call(
        _final_body,
        out_shape=jax.ShapeDtypeStruct((N_NODES, D), jnp.float32),
    )(aggp, x, W, b.reshape(1, D), gamma.reshape(1, D), beta.reshape(1, D))


def kernel(x, edge_index, edge_w, Wp1, Wp2, W, b, gamma, beta):
    src = edge_index[0]
    dst = edge_index[1]
    e = src.shape[0]
    span = CHUNK * NS * 2 * SUP
    e_pad = ((e + span - 1) // span) * span

    pad = e_pad - e
    src_p = jnp.concatenate([src, jnp.zeros((pad,), src.dtype)])
    dst_p = jnp.concatenate([dst, jnp.full((pad,), N_NODES, dst.dtype)])
    ewt_p = jnp.concatenate(
        [edge_w.T, jnp.zeros((2, pad), edge_w.dtype)], axis=1)
    perm = [c2 * DH + g2 * 32 + (t // 2 if t % 2 == 0 else 16 + t // 2)
            for c2 in range(2) for g2 in range(DH // 32) for t in range(32)]
    qlow = jnp.array(perm[0::2], dtype=jnp.int32)
    qhigh = jnp.array(perm[1::2], dtype=jnp.int32)
    xlo = lax.bitcast_convert_type(
        x[:, qlow].astype(jnp.bfloat16), jnp.uint16).astype(jnp.uint32)
    xhi = lax.bitcast_convert_type(
        x[:, qhigh].astype(jnp.bfloat16), jnp.uint16).astype(jnp.uint32)
    xpi = lax.bitcast_convert_type((xhi << 16) | xlo, jnp.int32)
    x_split = jnp.stack([xpi[:, :DH // 2], xpi[:, DH // 2:]], axis=0)
    Wp2p = Wp2[:, jnp.concatenate([qlow, qhigh])]
    idx8 = jnp.concatenate([src_p.reshape(-1, SUP, CHUNK),
                            dst_p.reshape(-1, SUP, CHUNK)], axis=1)

    pos1 = _pos_call(ewt_p, Wp1, Wp2p, blk=4096)
    aggp = _sc_call(idx8, pos1, x_split)
    return _final_call(aggp, x, W, b, gamma, beta)


# speedup vs baseline: 3.2654x; 1.0777x over previous
import jax
import jax.numpy as jnp
from jax import lax
from jax.experimental import pallas as pl
from jax.experimental.pallas import tpu as pltpu
from jax.experimental.pallas import tpu_sc as plsc

N_NODES = 10000
D = 128
DH = D // 2

NC = 2
NS = 16
LANES = 16

CHUNK = 128
NPAD = 10240
ZB = NPAD // NS
ZREP = ZB // CHUNK


def _pack_words(pv):
    lo = lax.bitcast_convert_type(
        pv[:, :DH].astype(jnp.bfloat16), jnp.uint16).astype(jnp.uint32)
    hi = lax.bitcast_convert_type(
        pv[:, DH:].astype(jnp.bfloat16), jnp.uint16).astype(jnp.uint32)
    return lax.bitcast_convert_type((hi << 16) | lo, jnp.int32)


def _pos_body(ewt1_ref, ewt2_ref, wp1_ref, wp2_ref, out_ref):
    def half(ewt_ref):
        ew = ewt_ref[...].T
        h = jnp.maximum(
            jnp.dot(ew, wp1_ref[...], preferred_element_type=jnp.float32),
            0.0)
        return _pack_words(
            jnp.dot(h, wp2_ref[...], preferred_element_type=jnp.float32)
            + 1.0)
    out_ref[...] = jnp.concatenate([half(ewt1_ref), half(ewt2_ref)], axis=1)


def _pos_call(ewt_pad, Wp1, Wp2, blk):
    e_pad = ewt_pad.shape[1]
    nb2 = e_pad // 2 // blk
    return pl.pallas_call(
        _pos_body,
        grid=(nb2,),
        in_specs=[
            pl.BlockSpec((2, blk), lambda i: (0, i)),
            pl.BlockSpec((2, blk), lambda i: (0, i + nb2)),
            pl.BlockSpec((2, D), lambda i: (0, 0)),
            pl.BlockSpec((D, D), lambda i: (0, 0)),
        ],
        out_specs=pl.BlockSpec((blk, D), lambda i: (i, 0)),
        out_shape=jax.ShapeDtypeStruct((e_pad // 2, D), jnp.int32),
    )(ewt_pad, ewt_pad, Wp1, Wp2)


SUP = 8
DW = DH + LANES


def _sc_body(idx_hbm, pos_hbm, x_hbm,
             agg_out,
             isup0_v, isup1_v,
             pos0_v, pos1_v,
             xg0_v, xg1_v, xg2_v, xg3_v,
             msg0_v, msg1_v, agg_sh,
             sem_i0, sem_i1,
             sem_p0, sem_p1,
             sem_g0, sem_g1, sem_g2, sem_g3,
             sem_a0, sem_a1):
    c = lax.axis_index("c")
    s = lax.axis_index("s")
    isupb = (isup0_v, isup1_v)
    posb = (pos0_v, pos1_v)
    xgb = (xg0_v, xg1_v, xg2_v, xg3_v)
    msgb = (msg0_v, msg1_v)
    sem_i = (sem_i0, sem_i1)
    sem_p = (sem_p0, sem_p1)
    sem_g = (sem_g0, sem_g1, sem_g2, sem_g3)
    sem_a = (sem_a0, sem_a1)

    def _fill_zero(i, _):
        for k in range(DW // LANES):
            msg0_v[i, pl.ds(k * LANES, LANES)] = jnp.zeros((LANES,), jnp.float32)
        return 0
    lax.fori_loop(0, CHUNK, _fill_zero, 0)

    for r in range(ZREP):
        row0 = s * ZB + r * CHUNK
        pltpu.sync_copy(msg0_v, agg_sh.at[pl.ds(row0, CHUNK)])

    def _fill_ones(i, _):
        msg0_v[i, pl.ds(DH, LANES)] = jnp.full((LANES,), 1.0, jnp.float32)
        msg1_v[i, pl.ds(DH, LANES)] = jnp.full((LANES,), 1.0, jnp.float32)
        return 0
    lax.fori_loop(0, CHUNK, _fill_ones, 0)
    plsc.subcore_barrier()

    nsup_tot = idx_hbm.shape[0]
    nsup = nsup_tot // NS
    cpt = nsup * SUP
    base_g = s * nsup
    base_c = base_g * SUP
    x_c = x_hbm.at[c]
    upper = base_c * CHUNK >= pos_hbm.shape[0]
    row_off = jnp.where(upper, pos_hbm.shape[0], 0)
    col0 = c * (DH // 2) + jnp.where(upper, DH, 0)

    def _load_super(slot, g):
        pltpu.async_copy(idx_hbm.at[g], isupb[slot], sem_i[slot])

    def _wait_super(slot, g):
        pltpu.make_async_copy(idx_hbm.at[g], isupb[slot], sem_i[slot]).wait()

    def _issue_pos(pslot, ci):
        pltpu.async_copy(
            pos_hbm.at[pl.ds(ci * CHUNK - row_off, CHUNK),
                       pl.ds(col0, DH // 2)],
            posb[pslot], sem_p[pslot])

    def _wait_pos(pslot, ci):
        pltpu.make_async_copy(
            pos_hbm.at[pl.ds(ci * CHUNK - row_off, CHUNK),
                       pl.ds(col0, DH // 2)],
            posb[pslot], sem_p[pslot]).wait()

    def _issue_gather(gslot, src_ref):
        pltpu.async_copy(x_c.at[src_ref], xgb[gslot], sem_g[gslot])

    def _wait_gather(gslot, src_ref):
        pltpu.make_async_copy(x_c.at[src_ref], xgb[gslot], sem_g[gslot]).wait()

    def _issue_scatter(mslot, dst_ref):
        pltpu.async_copy(msgb[mslot], agg_sh.at[dst_ref], sem_a[mslot],
                         add=True)

    def _wait_scatter(mslot, dst_ref):
        pltpu.make_async_copy(msgb[mslot], agg_sh.at[dst_ref],
                              sem_a[mslot]).wait()

    _load_super(0, base_g)
    _wait_super(0, base_g)
    _issue_gather(0, isup0_v.at[0])
    _issue_gather(1, isup0_v.at[1])
    _issue_pos(0, base_c)

    def _spair(g0, _):
        for gg in range(2):
            jbase = (g0 * 2 + gg) * SUP
            sg = gg
            nsg = 1 - gg
            for u in range(SUP):
                j = jbase + u
                ci = base_c + j
                gslot = u % 4
                mslot = u % 2
                src_j = isupb[sg].at[u]
                dst_j = isupb[sg].at[SUP + u]

                if u == 2:
                    @pl.when(j + SUP < cpt)
                    def _():
                        _load_super(nsg, base_g + (g0 * 2 + gg) + 1)
                if u == 6:
                    @pl.when(j + 2 < cpt)
                    def _():
                        _wait_super(nsg, base_g + (g0 * 2 + gg) + 1)

                if u < 6:
                    sref = isupb[sg].at[u + 2]
                else:
                    sref = isupb[nsg].at[u - 6]

                @pl.when(j + 2 < cpt)
                def _():
                    _issue_gather((u + 2) % 4, sref)

                @pl.when(j + 1 < cpt)
                def _():
                    _issue_pos((u + 1) % 2, ci + 1)

                if u < 2:
                    pdst = isupb[nsg].at[SUP + u + 6]

                    @pl.when(j >= 2)
                    def _():
                        _wait_scatter(mslot, pdst)
                else:
                    _wait_scatter(mslot, isupb[sg].at[SUP + u - 2])

                _wait_pos(mslot, ci)
                _wait_gather(gslot, src_j)

                @plsc.parallel_loop(0, CHUNK, step=1, unroll=8)
                def _(i):
                    for g2 in range(DH // 32):
                        wp = posb[mslot][i, pl.ds(g2 * LANES, LANES)]
                        wx = xgb[gslot][i, pl.ds(g2 * LANES, LANES)]
                        pe = lax.bitcast_convert_type(wp << 16, jnp.float32)
                        po = lax.bitcast_convert_type(
                            wp & jnp.int32(-65536), jnp.float32)
                        xe = lax.bitcast_convert_type(wx << 16, jnp.float32)
                        xo = lax.bitcast_convert_type(
                            wx & jnp.int32(-65536), jnp.float32)
                        msgb[mslot][i, pl.ds(g2 * 32, LANES)] = pe * xe
                        msgb[mslot][i, pl.ds(g2 * 32 + LANES, LANES)] = po * xo

                _issue_scatter(mslot, dst_j)
        return 0
    lax.fori_loop(0, nsup // 2, _spair, 0)

    _wait_scatter(0, isup1_v.at[SUP + 6])
    _wait_scatter(1, isup1_v.at[SUP + 7])
    plsc.subcore_barrier()

    for r in range(ZREP):
        row0 = s * ZB + r * CHUNK
        pltpu.sync_copy(agg_sh.at[pl.ds(row0, CHUNK)],
                        agg_out.at[c, pl.ds(row0, CHUNK)])


def _sc_call(idx8, pos1, x_split):
    mesh = plsc.VectorSubcoreMesh(core_axis_name="c", subcore_axis_name="s")
    f = pl.kernel(
        _sc_body,
        out_type=jax.ShapeDtypeStruct((NC, NPAD, DW), jnp.float32),
        mesh=mesh,
        compiler_params=pltpu.CompilerParams(use_tc_tiling_on_sc=False),
        scratch_types=(
            [pltpu.VMEM((2 * SUP, CHUNK), jnp.int32)] * 2
            + [pltpu.VMEM((CHUNK, DH // 2), jnp.int32)] * 2
            + [pltpu.VMEM((CHUNK, DH // 2), jnp.int32)] * 4
            + [pltpu.VMEM((CHUNK, DW), jnp.float32)] * 2
            + [pltpu.VMEM_SHARED((NPAD, DW), jnp.float32)]
            + [pltpu.SemaphoreType.DMA] * 10
        ),
    )
    return f(idx8, pos1, x_split)


def _final_body(p_ref, x_ref, w_ref, b_ref, g_ref, be_ref, out_ref):
    xv = x_ref[...]
    agg = jnp.concatenate(
        [p_ref[0, :N_NODES, :DH], p_ref[1, :N_NODES, :DH]], axis=1) + 2.0 * xv
    cnt = p_ref[0, :N_NODES, DH:DH + 1] + 1.0
    agg = agg / cnt
    o = jnp.dot(agg, w_ref[...], preferred_element_type=jnp.float32) + b_ref[...]
    mean = jnp.mean(o, axis=0, keepdims=True)
    var = jnp.mean((o - mean) * (o - mean), axis=0, keepdims=True)
    o = (o - mean) * lax.rsqrt(var + 1e-5) * g_ref[...] + be_ref[...]
    out_ref[...] = jnp.maximum(o, 0.0) + xv


def _final_call(aggp, x, W, b, gamma, beta):
    return pl.pallas_call(
        _final_body,
        out_shape=jax.ShapeDtypeStruct((N_NODES, D), jnp.float32),
    )(aggp, x, W, b.reshape(1, D), gamma.reshape(1, D), beta.reshape(1, D))


def kernel(x, edge_index, edge_w, Wp1, Wp2, W, b, gamma, beta):
    src = edge_index[0]
    dst = edge_index[1]
    e = src.shape[0]
    span = CHUNK * NS * 2 * SUP
    e_pad = ((e + span - 1) // span) * span

    pad = e_pad - e
    src_p = jnp.concatenate([src, jnp.zeros((pad,), src.dtype)])
    dst_p = jnp.concatenate([dst, jnp.full((pad,), N_NODES, dst.dtype)])
    ewt_p = jnp.concatenate(
        [edge_w.T, jnp.zeros((2, pad), edge_w.dtype)], axis=1)
    perm = [c2 * DH + g2 * 32 + (t // 2 if t % 2 == 0 else 16 + t // 2)
            for c2 in range(2) for g2 in range(DH // 32) for t in range(32)]
    qlow = jnp.array(perm[0::2], dtype=jnp.int32)
    qhigh = jnp.array(perm[1::2], dtype=jnp.int32)
    xlo = lax.bitcast_convert_type(
        x[:, qlow].astype(jnp.bfloat16), jnp.uint16).astype(jnp.uint32)
    xhi = lax.bitcast_convert_type(
        x[:, qhigh].astype(jnp.bfloat16), jnp.uint16).astype(jnp.uint32)
    xpi = lax.bitcast_convert_type((xhi << 16) | xlo, jnp.int32)
    x_split = jnp.stack([xpi[:, :DH // 2], xpi[:, DH // 2:]], axis=0)
    Wp2p = Wp2[:, jnp.concatenate([qlow, qhigh])]
    idx8 = jnp.concatenate([src_p.reshape(-1, SUP, CHUNK),
                            dst_p.reshape(-1, SUP, CHUNK)], axis=1)

    pos1 = _pos_call(ewt_p, Wp1, Wp2p, blk=4096)
    aggp = _sc_call(idx8, pos1, x_split)
    return _final_call(aggp, x, W, b, gamma, beta)


# speedup vs baseline: 3.3730x; 1.0330x over previous
import jax
import jax.numpy as jnp
from jax import lax
from jax.experimental import pallas as pl
from jax.experimental.pallas import tpu as pltpu
from jax.experimental.pallas import tpu_sc as plsc

N_NODES = 10000
D = 128
DH = D // 2

NC = 2
NS = 16
LANES = 16

CHUNK = 128
NPAD = 10240
ZB = NPAD // NS
ZREP = ZB // CHUNK


def _pack_words(pv):
    lo = lax.bitcast_convert_type(
        pv[:, :DH].astype(jnp.bfloat16), jnp.uint16).astype(jnp.uint32)
    hi = lax.bitcast_convert_type(
        pv[:, DH:].astype(jnp.bfloat16), jnp.uint16).astype(jnp.uint32)
    return lax.bitcast_convert_type((hi << 16) | lo, jnp.int32)


def _pos_body(ewt1_ref, ewt2_ref, wp1_ref, wp2_ref, out_ref):
    def half(ewt_ref):
        ew = ewt_ref[...].T.astype(jnp.bfloat16)
        h = jnp.maximum(
            jnp.dot(ew, wp1_ref[...].astype(jnp.bfloat16),
                    preferred_element_type=jnp.float32), 0.0)
        return _pack_words(
            jnp.dot(h.astype(jnp.bfloat16), wp2_ref[...].astype(jnp.bfloat16),
                    preferred_element_type=jnp.float32) + 1.0)
    out_ref[...] = jnp.concatenate([half(ewt1_ref), half(ewt2_ref)], axis=1)


def _pos_call(ewt_pad, Wp1, Wp2, blk):
    e_pad = ewt_pad.shape[1]
    nb2 = e_pad // 2 // blk
    return pl.pallas_call(
        _pos_body,
        grid=(nb2,),
        in_specs=[
            pl.BlockSpec((2, blk), lambda i: (0, i)),
            pl.BlockSpec((2, blk), lambda i: (0, i + nb2)),
            pl.BlockSpec((2, D), lambda i: (0, 0)),
            pl.BlockSpec((D, D), lambda i: (0, 0)),
        ],
        out_specs=pl.BlockSpec((blk, D), lambda i: (i, 0)),
        out_shape=jax.ShapeDtypeStruct((e_pad // 2, D), jnp.int32),
    )(ewt_pad, ewt_pad, Wp1, Wp2)


SUP = 8
DW = DH + LANES


def _sc_body(idx_hbm, pos_hbm, x_hbm,
             agg_out,
             isup0_v, isup1_v,
             pos0_v, pos1_v,
             xg0_v, xg1_v, xg2_v, xg3_v,
             msg0_v, msg1_v, msg2_v, msg3_v, agg_sh,
             sem_i0, sem_i1,
             sem_p0, sem_p1,
             sem_g0, sem_g1, sem_g2, sem_g3,
             sem_a0, sem_a1, sem_a2, sem_a3):
    c = lax.axis_index("c")
    s = lax.axis_index("s")
    isupb = (isup0_v, isup1_v)
    posb = (pos0_v, pos1_v)
    xgb = (xg0_v, xg1_v, xg2_v, xg3_v)
    msgb = (msg0_v, msg1_v, msg2_v, msg3_v)
    sem_i = (sem_i0, sem_i1)
    sem_p = (sem_p0, sem_p1)
    sem_g = (sem_g0, sem_g1, sem_g2, sem_g3)
    sem_a = (sem_a0, sem_a1, sem_a2, sem_a3)

    def _fill_zero(i, _):
        for k in range(DW // LANES):
            msg0_v[i, pl.ds(k * LANES, LANES)] = jnp.zeros((LANES,), jnp.float32)
        return 0
    lax.fori_loop(0, CHUNK, _fill_zero, 0)

    for r in range(ZREP):
        row0 = s * ZB + r * CHUNK
        pltpu.sync_copy(msg0_v, agg_sh.at[pl.ds(row0, CHUNK)])

    def _fill_ones(i, _):
        msg0_v[i, pl.ds(DH, LANES)] = jnp.full((LANES,), 1.0, jnp.float32)
        msg1_v[i, pl.ds(DH, LANES)] = jnp.full((LANES,), 1.0, jnp.float32)
        msg2_v[i, pl.ds(DH, LANES)] = jnp.full((LANES,), 1.0, jnp.float32)
        msg3_v[i, pl.ds(DH, LANES)] = jnp.full((LANES,), 1.0, jnp.float32)
        return 0
    lax.fori_loop(0, CHUNK, _fill_ones, 0)
    plsc.subcore_barrier()

    nsup_tot = idx_hbm.shape[0]
    nsup = nsup_tot // NS
    cpt = nsup * SUP
    base_g = s * nsup
    base_c = base_g * SUP
    x_c = x_hbm.at[c]
    upper = base_c * CHUNK >= pos_hbm.shape[0]
    row_off = jnp.where(upper, pos_hbm.shape[0], 0)
    col0 = c * (DH // 2) + jnp.where(upper, DH, 0)

    def _load_super(slot, g):
        pltpu.async_copy(idx_hbm.at[g], isupb[slot], sem_i[slot])

    def _wait_super(slot, g):
        pltpu.make_async_copy(idx_hbm.at[g], isupb[slot], sem_i[slot]).wait()

    def _issue_pos(pslot, ci):
        pltpu.async_copy(
            pos_hbm.at[pl.ds(ci * CHUNK - row_off, CHUNK),
                       pl.ds(col0, DH // 2)],
            posb[pslot], sem_p[pslot])

    def _wait_pos(pslot, ci):
        pltpu.make_async_copy(
            pos_hbm.at[pl.ds(ci * CHUNK - row_off, CHUNK),
                       pl.ds(col0, DH // 2)],
            posb[pslot], sem_p[pslot]).wait()

    def _issue_gather(gslot, src_ref):
        pltpu.async_copy(x_c.at[src_ref], xgb[gslot], sem_g[gslot])

    def _wait_gather(gslot, src_ref):
        pltpu.make_async_copy(x_c.at[src_ref], xgb[gslot], sem_g[gslot]).wait()

    def _issue_scatter(mslot, dst_ref):
        pltpu.async_copy(msgb[mslot], agg_sh.at[dst_ref], sem_a[mslot],
                         add=True)

    def _wait_scatter(mslot, dst_ref):
        pltpu.make_async_copy(msgb[mslot], agg_sh.at[dst_ref],
                              sem_a[mslot]).wait()

    _load_super(0, base_g)
    _wait_super(0, base_g)
    _issue_gather(0, isup0_v.at[0])
    _issue_gather(1, isup0_v.at[1])
    _issue_pos(0, base_c)

    def _spair(g0, _):
        for gg in range(2):
            jbase = (g0 * 2 + gg) * SUP
            sg = gg
            nsg = 1 - gg
            for u in range(SUP):
                j = jbase + u
                ci = base_c + j
                gslot = u % 4
                mslot = u % 4
                pslot = u % 2
                src_j = isupb[sg].at[u]
                dst_j = isupb[sg].at[SUP + u]

                if u == 4:
                    @pl.when(j + SUP < cpt)
                    def _():
                        _load_super(nsg, base_g + (g0 * 2 + gg) + 1)
                if u == 6:
                    @pl.when(j + 2 < cpt)
                    def _():
                        _wait_super(nsg, base_g + (g0 * 2 + gg) + 1)

                if u < 6:
                    sref = isupb[sg].at[u + 2]
                else:
                    sref = isupb[nsg].at[u - 6]

                @pl.when(j + 2 < cpt)
                def _():
                    _issue_gather((u + 2) % 4, sref)

                @pl.when(j + 1 < cpt)
                def _():
                    _issue_pos((u + 1) % 2, ci + 1)

                if u < 4:
                    pdst = isupb[nsg].at[SUP + u + 4]

                    @pl.when(j >= 4)
                    def _():
                        _wait_scatter(mslot, pdst)
                else:
                    _wait_scatter(mslot, isupb[sg].at[SUP + u - 4])

                _wait_pos(pslot, ci)
                _wait_gather(gslot, src_j)

                @plsc.parallel_loop(0, CHUNK, step=1, unroll=8)
                def _(i):
                    for g2 in range(DH // 32):
                        wp = posb[pslot][i, pl.ds(g2 * LANES, LANES)]
                        wx = xgb[gslot][i, pl.ds(g2 * LANES, LANES)]
                        pe = lax.bitcast_convert_type(wp << 16, jnp.float32)
                        po = lax.bitcast_convert_type(
                            wp & jnp.int32(-65536), jnp.float32)
                        xe = lax.bitcast_convert_type(wx << 16, jnp.float32)
                        xo = lax.bitcast_convert_type(
                            wx & jnp.int32(-65536), jnp.float32)
                        msgb[mslot][i, pl.ds(g2 * 32, LANES)] = pe * xe
                        msgb[mslot][i, pl.ds(g2 * 32 + LANES, LANES)] = po * xo

                _issue_scatter(mslot, dst_j)
        return 0
    lax.fori_loop(0, nsup // 2, _spair, 0)

    _wait_scatter(0, isup1_v.at[SUP + 4])
    _wait_scatter(1, isup1_v.at[SUP + 5])
    _wait_scatter(2, isup1_v.at[SUP + 6])
    _wait_scatter(3, isup1_v.at[SUP + 7])
    plsc.subcore_barrier()

    for r in range(ZREP):
        row0 = s * ZB + r * CHUNK
        pltpu.sync_copy(agg_sh.at[pl.ds(row0, CHUNK)],
                        agg_out.at[c, pl.ds(row0, CHUNK)])


def _sc_call(idx8, pos1, x_split):
    mesh = plsc.VectorSubcoreMesh(core_axis_name="c", subcore_axis_name="s")
    f = pl.kernel(
        _sc_body,
        out_type=jax.ShapeDtypeStruct((NC, NPAD, DW), jnp.float32),
        mesh=mesh,
        compiler_params=pltpu.CompilerParams(use_tc_tiling_on_sc=False),
        scratch_types=(
            [pltpu.VMEM((2 * SUP, CHUNK), jnp.int32)] * 2
            + [pltpu.VMEM((CHUNK, DH // 2), jnp.int32)] * 2
            + [pltpu.VMEM((CHUNK, DH // 2), jnp.int32)] * 4
            + [pltpu.VMEM((CHUNK, DW), jnp.float32)] * 4
            + [pltpu.VMEM_SHARED((NPAD, DW), jnp.float32)]
            + [pltpu.SemaphoreType.DMA] * 12
        ),
    )
    return f(idx8, pos1, x_split)


def _final_body(p_ref, x_ref, w_ref, b_ref, g_ref, be_ref, out_ref):
    xv = x_ref[...]
    agg = jnp.concatenate(
        [p_ref[0, :N_NODES, :DH], p_ref[1, :N_NODES, :DH]], axis=1) + 2.0 * xv
    cnt = p_ref[0, :N_NODES, DH:DH + 1] + 1.0
    agg = agg / cnt
    o = jnp.dot(agg, w_ref[...], preferred_element_type=jnp.float32) + b_ref[...]
    mean = jnp.mean(o, axis=0, keepdims=True)
    var = jnp.mean((o - mean) * (o - mean), axis=0, keepdims=True)
    o = (o - mean) * lax.rsqrt(var + 1e-5) * g_ref[...] + be_ref[...]
    out_ref[...] = jnp.maximum(o, 0.0) + xv


def _final_call(aggp, x, W, b, gamma, beta):
    return pl.pallas_call(
        _final_body,
        out_shape=jax.ShapeDtypeStruct((N_NODES, D), jnp.float32),
    )(aggp, x, W, b.reshape(1, D), gamma.reshape(1, D), beta.reshape(1, D))


def kernel(x, edge_index, edge_w, Wp1, Wp2, W, b, gamma, beta):
    src = edge_index[0]
    dst = edge_index[1]
    e = src.shape[0]
    span = CHUNK * NS * 2 * SUP
    e_pad = ((e + span - 1) // span) * span

    pad = e_pad - e
    src_p = jnp.concatenate([src, jnp.zeros((pad,), src.dtype)])
    dst_p = jnp.concatenate([dst, jnp.full((pad,), N_NODES, dst.dtype)])
    ewt_p = jnp.concatenate(
        [edge_w.T, jnp.zeros((2, pad), edge_w.dtype)], axis=1)
    perm = [c2 * DH + g2 * 32 + (t // 2 if t % 2 == 0 else 16 + t // 2)
            for c2 in range(2) for g2 in range(DH // 32) for t in range(32)]
    qlow = jnp.array(perm[0::2], dtype=jnp.int32)
    qhigh = jnp.array(perm[1::2], dtype=jnp.int32)
    xlo = lax.bitcast_convert_type(
        x[:, qlow].astype(jnp.bfloat16), jnp.uint16).astype(jnp.uint32)
    xhi = lax.bitcast_convert_type(
        x[:, qhigh].astype(jnp.bfloat16), jnp.uint16).astype(jnp.uint32)
    xpi = lax.bitcast_convert_type((xhi << 16) | xlo, jnp.int32)
    x_split = jnp.stack([xpi[:, :DH // 2], xpi[:, DH // 2:]], axis=0)
    Wp2p = Wp2[:, jnp.concatenate([qlow, qhigh])]
    idx8 = jnp.concatenate([src_p.reshape(-1, SUP, CHUNK),
                            dst_p.reshape(-1, SUP, CHUNK)], axis=1)

    pos1 = _pos_call(ewt_p, Wp1, Wp2p, blk=4096)
    aggp = _sc_call(idx8, pos1, x_split)
    return _final_call(aggp, x, W, b, gamma, beta)


# speedup vs baseline: 3.4236x; 1.0150x over previous
import jax
import jax.numpy as jnp
from jax import lax
from jax.experimental import pallas as pl
from jax.experimental.pallas import tpu as pltpu
from jax.experimental.pallas import tpu_sc as plsc

N_NODES = 10000
D = 128
DH = D // 2

NC = 2
NS = 16
LANES = 16

CHUNK = 128
NPAD = 10240
ZB = NPAD // NS
ZREP = ZB // CHUNK


def _pack_words(pv):
    lo = lax.bitcast_convert_type(
        pv[:, :DH].astype(jnp.bfloat16), jnp.uint16).astype(jnp.uint32)
    hi = lax.bitcast_convert_type(
        pv[:, DH:].astype(jnp.bfloat16), jnp.uint16).astype(jnp.uint32)
    return lax.bitcast_convert_type((hi << 16) | lo, jnp.int32)


def _pos_body(ewt1_ref, ewt2_ref, wp1_ref, wp2_ref, out_ref):
    def half(ewt_ref):
        ew = ewt_ref[...].T.astype(jnp.bfloat16)
        h = jnp.maximum(
            jnp.dot(ew, wp1_ref[...].astype(jnp.bfloat16),
                    preferred_element_type=jnp.float32), 0.0)
        return _pack_words(
            jnp.dot(h.astype(jnp.bfloat16), wp2_ref[...].astype(jnp.bfloat16),
                    preferred_element_type=jnp.float32) + 1.0)
    out_ref[...] = jnp.concatenate([half(ewt1_ref), half(ewt2_ref)], axis=1)


def _pos_call(ewt_pad, Wp1, Wp2, blk):
    e_pad = ewt_pad.shape[1]
    nb2 = e_pad // 2 // blk
    return pl.pallas_call(
        _pos_body,
        grid=(nb2,),
        in_specs=[
            pl.BlockSpec((2, blk), lambda i: (0, i)),
            pl.BlockSpec((2, blk), lambda i: (0, i + nb2)),
            pl.BlockSpec((2, D), lambda i: (0, 0)),
            pl.BlockSpec((D, D), lambda i: (0, 0)),
        ],
        out_specs=pl.BlockSpec((blk, D), lambda i: (i, 0)),
        out_shape=jax.ShapeDtypeStruct((e_pad // 2, D), jnp.int32),
    )(ewt_pad, ewt_pad, Wp1, Wp2)


SUP = 8
DW = DH + LANES


def _sc_body(idx_hbm, pos_hbm, x_hbm,
             agg_out,
             isup0_v, isup1_v,
             pos0_v, pos1_v,
             xg0_v, xg1_v, xg2_v, xg3_v,
             msg0_v, msg1_v, msg2_v, msg3_v, agg_sh,
             sem_i0, sem_i1,
             sem_p0, sem_p1,
             sem_g0, sem_g1, sem_g2, sem_g3,
             sem_a0, sem_a1, sem_a2, sem_a3):
    c = lax.axis_index("c")
    s = lax.axis_index("s")
    isupb = (isup0_v, isup1_v)
    posb = (pos0_v, pos1_v)
    xgb = (xg0_v, xg1_v, xg2_v, xg3_v)
    msgb = (msg0_v, msg1_v, msg2_v, msg3_v)
    sem_i = (sem_i0, sem_i1)
    sem_p = (sem_p0, sem_p1)
    sem_g = (sem_g0, sem_g1, sem_g2, sem_g3)
    sem_a = (sem_a0, sem_a1, sem_a2, sem_a3)

    def _fill_zero(i, _):
        for k in range(DW // LANES):
            msg0_v[i, pl.ds(k * LANES, LANES)] = jnp.zeros((LANES,), jnp.float32)
        return 0
    lax.fori_loop(0, CHUNK, _fill_zero, 0)

    for r in range(ZREP):
        row0 = s * ZB + r * CHUNK
        pltpu.sync_copy(msg0_v, agg_sh.at[pl.ds(row0, CHUNK)])

    def _fill_ones(i, _):
        msg0_v[i, pl.ds(DH, LANES)] = jnp.full((LANES,), 1.0, jnp.float32)
        msg1_v[i, pl.ds(DH, LANES)] = jnp.full((LANES,), 1.0, jnp.float32)
        msg2_v[i, pl.ds(DH, LANES)] = jnp.full((LANES,), 1.0, jnp.float32)
        msg3_v[i, pl.ds(DH, LANES)] = jnp.full((LANES,), 1.0, jnp.float32)
        return 0
    lax.fori_loop(0, CHUNK, _fill_ones, 0)
    plsc.subcore_barrier()

    nsup_tot = idx_hbm.shape[0]
    nsup = nsup_tot // NS
    cpt = nsup * SUP
    base_g = s * nsup
    base_c = base_g * SUP
    x_c = x_hbm.at[c]
    upper = base_c * CHUNK >= pos_hbm.shape[0]
    row_off = jnp.where(upper, pos_hbm.shape[0], 0)
    col0 = c * (DH // 2) + jnp.where(upper, DH, 0)

    def _load_super(slot, g):
        pltpu.async_copy(idx_hbm.at[g], isupb[slot], sem_i[slot])

    def _wait_super(slot, g):
        pltpu.make_async_copy(idx_hbm.at[g], isupb[slot], sem_i[slot]).wait()

    def _issue_pos(pslot, ci):
        pltpu.async_copy(
            pos_hbm.at[pl.ds(ci * CHUNK - row_off, CHUNK),
                       pl.ds(col0, DH // 2)],
            posb[pslot], sem_p[pslot])

    def _wait_pos(pslot, ci):
        pltpu.make_async_copy(
            pos_hbm.at[pl.ds(ci * CHUNK - row_off, CHUNK),
                       pl.ds(col0, DH // 2)],
            posb[pslot], sem_p[pslot]).wait()

    def _issue_gather(gslot, src_ref):
        pltpu.async_copy(x_c.at[src_ref], xgb[gslot], sem_g[gslot])

    def _wait_gather(gslot, src_ref):
        pltpu.make_async_copy(x_c.at[src_ref], xgb[gslot], sem_g[gslot]).wait()

    def _issue_scatter(mslot, dst_ref):
        pltpu.async_copy(msgb[mslot], agg_sh.at[dst_ref], sem_a[mslot],
                         add=True)

    def _wait_scatter(mslot, dst_ref):
        pltpu.make_async_copy(msgb[mslot], agg_sh.at[dst_ref],
                              sem_a[mslot]).wait()

    _load_super(0, base_g)
    _wait_super(0, base_g)
    _issue_gather(0, isup0_v.at[0])
    _issue_gather(1, isup0_v.at[1])
    _issue_pos(0, base_c)

    def _spair(g0, _):
        for gg in range(2):
            jbase = (g0 * 2 + gg) * SUP
            sg = gg
            nsg = 1 - gg
            for u in range(SUP):
                j = jbase + u
                ci = base_c + j
                gslot = u % 4
                mslot = u % 4
                pslot = u % 2
                src_j = isupb[sg].at[u]
                dst_j = isupb[sg].at[SUP + u]

                if u == 4:
                    @pl.when(j + SUP < cpt)
                    def _():
                        _load_super(nsg, base_g + (g0 * 2 + gg) + 1)
                if u == 6:
                    @pl.when(j + 2 < cpt)
                    def _():
                        _wait_super(nsg, base_g + (g0 * 2 + gg) + 1)

                if u < 6:
                    sref = isupb[sg].at[u + 2]
                else:
                    sref = isupb[nsg].at[u - 6]

                @pl.when(j + 2 < cpt)
                def _():
                    _issue_gather((u + 2) % 4, sref)

                @pl.when(j + 1 < cpt)
                def _():
                    _issue_pos((u + 1) % 2, ci + 1)

                if u < 4:
                    pdst = isupb[nsg].at[SUP + u + 4]

                    @pl.when(j >= 4)
                    def _():
                        _wait_scatter(mslot, pdst)
                else:
                    _wait_scatter(mslot, isupb[sg].at[SUP + u - 4])

                _wait_pos(pslot, ci)
                _wait_gather(gslot, src_j)

                @plsc.parallel_loop(0, CHUNK, step=1, unroll=8)
                def _(i):
                    for g2 in range(DH // 32):
                        wp = posb[pslot][i, pl.ds(g2 * LANES, LANES)]
                        wx = xgb[gslot][i, pl.ds(g2 * LANES, LANES)]
                        pe = lax.bitcast_convert_type(wp << 16, jnp.float32)
                        po = lax.bitcast_convert_type(
                            wp & jnp.int32(-65536), jnp.float32)
                        xe = lax.bitcast_convert_type(wx << 16, jnp.float32)
                        xo = lax.bitcast_convert_type(
                            wx & jnp.int32(-65536), jnp.float32)
                        msgb[mslot][i, pl.ds(g2 * 32, LANES)] = pe * xe
                        msgb[mslot][i, pl.ds(g2 * 32 + LANES, LANES)] = po * xo

                _issue_scatter(mslot, dst_j)
        return 0
    lax.fori_loop(0, nsup // 2, _spair, 0)

    _wait_scatter(0, isup1_v.at[SUP + 4])
    _wait_scatter(1, isup1_v.at[SUP + 5])
    _wait_scatter(2, isup1_v.at[SUP + 6])
    _wait_scatter(3, isup1_v.at[SUP + 7])
    plsc.subcore_barrier()

    for r in range(ZREP):
        row0 = s * ZB + r * CHUNK
        pltpu.sync_copy(agg_sh.at[pl.ds(row0, CHUNK)],
                        agg_out.at[c, pl.ds(row0, CHUNK)])


def _sc_call(idx8, pos1, x_split):
    mesh = plsc.VectorSubcoreMesh(core_axis_name="c", subcore_axis_name="s")
    f = pl.kernel(
        _sc_body,
        out_type=jax.ShapeDtypeStruct((NC, NPAD, DW), jnp.float32),
        mesh=mesh,
        compiler_params=pltpu.CompilerParams(use_tc_tiling_on_sc=False),
        scratch_types=(
            [pltpu.VMEM((2 * SUP, CHUNK), jnp.int32)] * 2
            + [pltpu.VMEM((CHUNK, DH // 2), jnp.int32)] * 2
            + [pltpu.VMEM((CHUNK, DH // 2), jnp.int32)] * 4
            + [pltpu.VMEM((CHUNK, DW), jnp.float32)] * 4
            + [pltpu.VMEM_SHARED((NPAD, DW), jnp.float32)]
            + [pltpu.SemaphoreType.DMA] * 12
        ),
    )
    return f(idx8, pos1, x_split)


def _final_body(p_ref, x_ref, w_ref, b_ref, g_ref, be_ref, out_ref):
    xv = x_ref[...]
    agg = jnp.concatenate(
        [p_ref[0, :N_NODES, :DH], p_ref[1, :N_NODES, :DH]], axis=1) + 2.0 * xv
    cnt = p_ref[0, :N_NODES, DH:DH + 1] + 1.0
    agg = agg / cnt
    o = jnp.dot(agg, w_ref[...], preferred_element_type=jnp.float32) + b_ref[...]
    mean = jnp.mean(o, axis=0, keepdims=True)
    var = jnp.mean((o - mean) * (o - mean), axis=0, keepdims=True)
    o = (o - mean) * lax.rsqrt(var + 1e-5) * g_ref[...] + be_ref[...]
    out_ref[...] = jnp.maximum(o, 0.0) + xv


def _final_call(aggp, x, W, b, gamma, beta):
    return pl.pallas_call(
        _final_body,
        out_shape=jax.ShapeDtypeStruct((N_NODES, D), jnp.float32),
    )(aggp, x, W, b.reshape(1, D), gamma.reshape(1, D), beta.reshape(1, D))


def kernel(x, edge_index, edge_w, Wp1, Wp2, W, b, gamma, beta):
    src = edge_index[0]
    dst = edge_index[1]
    e = src.shape[0]
    span = CHUNK * NS * 2 * SUP
    e_pad = ((e + span - 1) // span) * span

    pad = e_pad - e
    src_p = jnp.concatenate([src, jnp.zeros((pad,), src.dtype)])
    dst_p = jnp.concatenate([dst, jnp.full((pad,), N_NODES, dst.dtype)])
    ewt_p = jnp.concatenate(
        [edge_w.T, jnp.zeros((2, pad), edge_w.dtype)], axis=1)
    perm = [c2 * DH + g2 * 32 + (t // 2 if t % 2 == 0 else 16 + t // 2)
            for c2 in range(2) for g2 in range(DH // 32) for t in range(32)]
    qlow = jnp.array(perm[0::2], dtype=jnp.int32)
    qhigh = jnp.array(perm[1::2], dtype=jnp.int32)
    xlo = lax.bitcast_convert_type(
        x[:, qlow].astype(jnp.bfloat16), jnp.uint16).astype(jnp.uint32)
    xhi = lax.bitcast_convert_type(
        x[:, qhigh].astype(jnp.bfloat16), jnp.uint16).astype(jnp.uint32)
    xpi = lax.bitcast_convert_type((xhi << 16) | xlo, jnp.int32)
    x_split = jnp.stack([xpi[:, :DH // 2], xpi[:, DH // 2:]], axis=0)
    Wp2p = Wp2[:, jnp.concatenate([qlow, qhigh])]
    idx8 = jnp.concatenate([src_p.reshape(-1, SUP, CHUNK),
                            dst_p.reshape(-1, SUP, CHUNK)], axis=1)

    pos1 = _pos_call(ewt_p, Wp1, Wp2p, blk=8192)
    aggp = _sc_call(idx8, pos1, x_split)
    return _final_call(aggp, x, W, b, gamma, beta)
